# Initial kernel scaffold; baseline (speedup 1.0000x reference)
import jax
import jax.numpy as jnp
from jax.experimental import pallas as pl


def kernel(x, W_base, embed_table, W_proj, init_A, init_B, input_ids):
    raise NotImplementedError("write your pallas kernel here")


# speedup vs baseline: 1.0984x; 1.0984x over previous
import jax
import jax.numpy as jnp
from jax.experimental import pallas as pl
from jax.experimental.pallas import tpu as pltpu

CHUNK = 64
TTT_LR = 0.01
LR_SCALE = 2.0
LR = TTT_LR * LR_SCALE

_ANY = getattr(pl, "ANY", None)
if _ANY is None:
    _ANY = pltpu.ANY


def _gather_v_call(ids_next, embed_table, wp_bf, BS, S, H, E):
    T = min(512, S)
    assert S % T == 0 and BS % T == 0
    bh = min(1024, H)
    assert H % bh == 0
    ratio = S // T

    def body(ids_ref, emb_ref, wp_ref, v_ref, gat_ref, sem):
        t = pl.program_id(0)
        ph = pl.program_id(1)

        @pl.when(ph == 0)
        def _():
            base = t * T
            for i in range(T):
                pltpu.make_async_copy(
                    emb_ref.at[ids_ref[base + i]], gat_ref.at[i], sem
                ).start()
            for i in range(T):
                pltpu.make_async_copy(
                    emb_ref.at[ids_ref[base + i]], gat_ref.at[i], sem
                ).wait()

            @pl.when((t + 1) % ratio == 0)
            def _():
                gat_ref[T - 1:T, :] = jnp.zeros((1, E), jnp.float32)

        v_ref[...] = jax.lax.dot_general(
            gat_ref[...].astype(jnp.bfloat16), wp_ref[...],
            (((1,), (1,)), ((), ())),
            preferred_element_type=jnp.float32,
        ).astype(jnp.bfloat16)

    return pl.pallas_call(
        body,
        grid=(BS // T, H // bh),
        in_specs=[
            pl.BlockSpec(memory_space=pltpu.SMEM),
            pl.BlockSpec(memory_space=_ANY),
            pl.BlockSpec((bh, E), lambda t, h: (h, 0)),
        ],
        out_specs=pl.BlockSpec((T, bh), lambda t, h: (t, h)),
        out_shape=jax.ShapeDtypeStruct((BS, H), jnp.bfloat16),
        scratch_shapes=[
            pltpu.VMEM((T, E), jnp.float32),
            pltpu.SemaphoreType.DMA,
        ],
        compiler_params=pltpu.CompilerParams(
            dimension_semantics=("parallel", "arbitrary"),
            vmem_limit_bytes=56 * 1024 * 1024,
        ),
        name="gather_v",
    )(ids_next, embed_table, wp_bf)


def _ttt_call(x_bf, v_bf, init_A, init_B, B, NC, K, H, r):
    def body(x_ref, v_ref, ia_ref, ib_ref, o_ref, accA, accB):
        n = pl.program_id(1)

        @pl.when(n == 0)
        def _():
            accA[...] = jnp.zeros_like(accA)
            accB[...] = jnp.zeros_like(accB)

        Zc = x_ref[...]
        Vc = v_ref[...]
        iA = ia_ref[...]
        iB = ib_ref[...]

        A_eff = (iA - LR * accA[...]).astype(jnp.bfloat16)
        B_eff = (iB - LR * accB[...]).astype(jnp.bfloat16)

        mid = jax.lax.dot_general(
            Zc, B_eff, (((1,), (1,)), ((), ())),
            preferred_element_type=jnp.float32)
        lora = jax.lax.dot_general(
            mid.astype(jnp.bfloat16), A_eff, (((1,), (1,)), ((), ())),
            preferred_element_type=jnp.float32)
        o_ref[...] = (lora * LR_SCALE).astype(jnp.bfloat16)

        proj_in = jax.lax.dot_general(
            Zc, iB.astype(jnp.bfloat16), (((1,), (1,)), ((), ())),
            preferred_element_type=jnp.float32)
        dA = jax.lax.dot_general(
            Vc, proj_in.astype(jnp.bfloat16), (((0,), (0,)), ((), ())),
            preferred_element_type=jnp.float32)
        accA[...] += dA

        proj_err = jax.lax.dot_general(
            Vc, iA.astype(jnp.bfloat16), (((1,), (0,)), ((), ())),
            preferred_element_type=jnp.float32)
        dB = jax.lax.dot_general(
            proj_err.astype(jnp.bfloat16), Zc, (((0,), (0,)), ((), ())),
            preferred_element_type=jnp.float32)
        accB[...] += dB

    return pl.pallas_call(
        body,
        grid=(B, NC),
        in_specs=[
            pl.BlockSpec((CHUNK, K), lambda b, n: (b * NC + n, 0)),
            pl.BlockSpec((CHUNK, H), lambda b, n: (b * NC + n, 0)),
            pl.BlockSpec((H, r), lambda b, n: (0, 0)),
            pl.BlockSpec((r, K), lambda b, n: (0, 0)),
        ],
        out_specs=pl.BlockSpec((CHUNK, H), lambda b, n: (b * NC + n, 0)),
        out_shape=jax.ShapeDtypeStruct((B * NC * CHUNK, H), jnp.bfloat16),
        scratch_shapes=[
            pltpu.VMEM((H, r), jnp.float32),
            pltpu.VMEM((r, K), jnp.float32),
        ],
        compiler_params=pltpu.CompilerParams(
            dimension_semantics=("parallel", "arbitrary"),
            vmem_limit_bytes=56 * 1024 * 1024,
        ),
        name="ttt_scan",
    )(x_bf, v_bf, init_A, init_B)


def _base_call(x_bf, wb_bf, lora_bf, BS, K, H):
    bm = min(512, BS)
    bn = min(512, H)
    assert BS % bm == 0 and H % bn == 0

    def body(x_ref, w_ref, l_ref, o_ref):
        o_ref[...] = jax.lax.dot_general(
            x_ref[...], w_ref[...], (((1,), (1,)), ((), ())),
            preferred_element_type=jnp.float32,
        ) + l_ref[...].astype(jnp.float32)

    return pl.pallas_call(
        body,
        grid=(BS // bm, H // bn),
        in_specs=[
            pl.BlockSpec((bm, K), lambda i, j: (i, 0)),
            pl.BlockSpec((bn, K), lambda i, j: (j, 0)),
            pl.BlockSpec((bm, bn), lambda i, j: (i, j)),
        ],
        out_specs=pl.BlockSpec((bm, bn), lambda i, j: (i, j)),
        out_shape=jax.ShapeDtypeStruct((BS, H), jnp.float32),
        compiler_params=pltpu.CompilerParams(
            dimension_semantics=("parallel", "arbitrary"),
            vmem_limit_bytes=56 * 1024 * 1024,
        ),
        name="base_lora",
    )(x_bf, wb_bf, lora_bf)


def kernel(x, W_base, embed_table, W_proj, init_A, init_B, input_ids):
    B, S, K = x.shape
    H = W_base.shape[0]
    E = embed_table.shape[1]
    r = init_A.shape[1]
    BS = B * S
    NC = S // CHUNK
    assert S % CHUNK == 0

    x_bf = x.reshape(BS, K).astype(jnp.bfloat16)
    wb_bf = W_base.astype(jnp.bfloat16)
    wp_bf = W_proj.astype(jnp.bfloat16)
    ids_next = jnp.concatenate(
        [input_ids[:, 1:], input_ids[:, :1]], axis=1
    ).reshape(BS).astype(jnp.int32)

    v_bf = _gather_v_call(ids_next, embed_table, wp_bf, BS, S, H, E)
    lora_bf = _ttt_call(x_bf, v_bf, init_A, init_B, B, NC, K, H, r)
    out = _base_call(x_bf, wb_bf, lora_bf, BS, K, H)
    return out.reshape(B, S, H)


# speedup vs baseline: 1.2174x; 1.1083x over previous
import jax
import jax.numpy as jnp
from jax.experimental import pallas as pl
from jax.experimental.pallas import tpu as pltpu

CHUNK = 64
TTT_LR = 0.01
LR_SCALE = 2.0
LR = TTT_LR * LR_SCALE

_ANY = getattr(pl, "ANY", None)
if _ANY is None:
    _ANY = pltpu.ANY


def _gather_v_call(ids_next, embed_table, wp_bf, BS, S, H, E):
    T = min(512, S)
    assert S % T == 0 and BS % T == 0
    bh = min(1024, H)
    assert H % bh == 0
    ratio = S // T

    def body(ids_ref, emb_ref, wp_ref, v_ref, gat_ref, sem):
        t = pl.program_id(0)
        ph = pl.program_id(1)

        @pl.when(ph == 0)
        def _():
            base = t * T
            for i in range(T):
                pltpu.make_async_copy(
                    emb_ref.at[ids_ref[base + i]], gat_ref.at[i], sem
                ).start()
            for i in range(T):
                pltpu.make_async_copy(
                    emb_ref.at[ids_ref[base + i]], gat_ref.at[i], sem
                ).wait()

            @pl.when((t + 1) % ratio == 0)
            def _():
                gat_ref[T - 1:T, :] = jnp.zeros((1, E), jnp.float32)

        v_ref[...] = jax.lax.dot_general(
            gat_ref[...].astype(jnp.bfloat16), wp_ref[...],
            (((1,), (1,)), ((), ())),
            preferred_element_type=jnp.float32,
        ).astype(jnp.bfloat16)

    return pl.pallas_call(
        body,
        grid=(BS // T, H // bh),
        in_specs=[
            pl.BlockSpec(memory_space=pltpu.SMEM),
            pl.BlockSpec(memory_space=_ANY),
            pl.BlockSpec((bh, E), lambda t, h: (h, 0)),
        ],
        out_specs=pl.BlockSpec((T, bh), lambda t, h: (t, h)),
        out_shape=jax.ShapeDtypeStruct((BS, H), jnp.bfloat16),
        scratch_shapes=[
            pltpu.VMEM((T, E), jnp.float32),
            pltpu.SemaphoreType.DMA,
        ],
        compiler_params=pltpu.CompilerParams(
            dimension_semantics=("parallel", "arbitrary"),
            vmem_limit_bytes=56 * 1024 * 1024,
        ),
        name="gather_v",
    )(ids_next, embed_table, wp_bf)


def _ttt_call(x_f32, v_bf, init_A, init_BT, B, NC, K, H, r):
    def body(x_ref, v_ref, ia_ref, ibt_ref, o_ref, xb_ref, accA, accB):
        n = pl.program_id(1)

        @pl.when(n == 0)
        def _():
            accA[...] = jnp.zeros_like(accA)
            accB[...] = jnp.zeros_like(accB)

        Zc = x_ref[...].astype(jnp.bfloat16)
        xb_ref[...] = Zc
        Vc = v_ref[...]
        iA = ia_ref[...]
        iBT = ibt_ref[...]

        A_eff = (iA - LR * accA[...]).astype(jnp.bfloat16)
        rhs2 = jnp.concatenate(
            [iBT.astype(jnp.bfloat16), accB[...].astype(jnp.bfloat16)], axis=1
        )
        p2 = jax.lax.dot_general(
            Zc, rhs2, (((1,), (0,)), ((), ())),
            preferred_element_type=jnp.float32)
        proj_in = p2[:, :r]
        mid = proj_in - LR * p2[:, r:]

        lora = jax.lax.dot_general(
            mid.astype(jnp.bfloat16), A_eff, (((1,), (1,)), ((), ())),
            preferred_element_type=jnp.float32)
        o_ref[...] = (lora * LR_SCALE).astype(jnp.bfloat16)

        dA = jax.lax.dot_general(
            Vc, proj_in.astype(jnp.bfloat16), (((0,), (0,)), ((), ())),
            preferred_element_type=jnp.float32)
        accA[...] += dA

        proj_err = jax.lax.dot_general(
            Vc, iA.astype(jnp.bfloat16), (((1,), (0,)), ((), ())),
            preferred_element_type=jnp.float32)
        dBT = jax.lax.dot_general(
            Zc, proj_err.astype(jnp.bfloat16), (((0,), (0,)), ((), ())),
            preferred_element_type=jnp.float32)
        accB[...] += dBT

    return pl.pallas_call(
        body,
        grid=(B, NC),
        in_specs=[
            pl.BlockSpec((CHUNK, K), lambda b, n: (b * NC + n, 0)),
            pl.BlockSpec((CHUNK, H), lambda b, n: (b * NC + n, 0)),
            pl.BlockSpec((H, r), lambda b, n: (0, 0)),
            pl.BlockSpec((K, r), lambda b, n: (0, 0)),
        ],
        out_specs=[
            pl.BlockSpec((CHUNK, H), lambda b, n: (b * NC + n, 0)),
            pl.BlockSpec((CHUNK, K), lambda b, n: (b * NC + n, 0)),
        ],
        out_shape=[
            jax.ShapeDtypeStruct((B * NC * CHUNK, H), jnp.bfloat16),
            jax.ShapeDtypeStruct((B * NC * CHUNK, K), jnp.bfloat16),
        ],
        scratch_shapes=[
            pltpu.VMEM((H, r), jnp.float32),
            pltpu.VMEM((K, r), jnp.float32),
        ],
        compiler_params=pltpu.CompilerParams(
            dimension_semantics=("parallel", "arbitrary"),
            vmem_limit_bytes=56 * 1024 * 1024,
        ),
        name="ttt_scan",
    )(x_f32, v_bf, init_A, init_BT)


def _base_call(x_bf, wb_bf, lora_bf, BS, K, H):
    bm = min(512, BS)
    bn = min(512, H)
    assert BS % bm == 0 and H % bn == 0

    def body(x_ref, w_ref, l_ref, o_ref):
        o_ref[...] = jax.lax.dot_general(
            x_ref[...], w_ref[...], (((1,), (1,)), ((), ())),
            preferred_element_type=jnp.float32,
        ) + l_ref[...].astype(jnp.float32)

    return pl.pallas_call(
        body,
        grid=(BS // bm, H // bn),
        in_specs=[
            pl.BlockSpec((bm, K), lambda i, j: (i, 0)),
            pl.BlockSpec((bn, K), lambda i, j: (j, 0)),
            pl.BlockSpec((bm, bn), lambda i, j: (i, j)),
        ],
        out_specs=pl.BlockSpec((bm, bn), lambda i, j: (i, j)),
        out_shape=jax.ShapeDtypeStruct((BS, H), jnp.float32),
        compiler_params=pltpu.CompilerParams(
            dimension_semantics=("parallel", "arbitrary"),
            vmem_limit_bytes=56 * 1024 * 1024,
        ),
        name="base_lora",
    )(x_bf, wb_bf, lora_bf)


def kernel(x, W_base, embed_table, W_proj, init_A, init_B, input_ids):
    B, S, K = x.shape
    H = W_base.shape[0]
    E = embed_table.shape[1]
    r = init_A.shape[1]
    BS = B * S
    NC = S // CHUNK
    assert S % CHUNK == 0

    wb_bf = W_base.astype(jnp.bfloat16)
    wp_bf = W_proj.astype(jnp.bfloat16)
    ids_next = jnp.concatenate(
        [input_ids[:, 1:], input_ids[:, :1]], axis=1
    ).reshape(BS).astype(jnp.int32)

    v_bf = _gather_v_call(ids_next, embed_table, wp_bf, BS, S, H, E)
    lora_bf, x_bf = _ttt_call(
        x.reshape(BS, K), v_bf, init_A, init_B.T, B, NC, K, H, r)
    out = _base_call(x_bf, wb_bf, lora_bf, BS, K, H)
    return out.reshape(B, S, H)


# speedup vs baseline: 1.2320x; 1.0119x over previous
import jax
import jax.numpy as jnp
from jax.experimental import pallas as pl
from jax.experimental.pallas import tpu as pltpu

CHUNK = 64
TTT_LR = 0.01
LR_SCALE = 2.0
LR = TTT_LR * LR_SCALE

_ANY = getattr(pl, "ANY", None)
if _ANY is None:
    _ANY = pltpu.ANY


def _gather_call(ids_next, embed_table, BS, S, E):
    T = min(512, S)
    assert S % T == 0 and BS % T == 0
    ratio = S // T

    def body(ids_ref, emb_ref, s_ref, gat_ref, sem):
        t = pl.program_id(0)
        base = t * T
        for i in range(T):
            pltpu.make_async_copy(
                emb_ref.at[ids_ref[base + i]], gat_ref.at[i], sem
            ).start()
        for i in range(T):
            pltpu.make_async_copy(
                emb_ref.at[ids_ref[base + i]], gat_ref.at[i], sem
            ).wait()

        @pl.when((t + 1) % ratio == 0)
        def _():
            gat_ref[T - 1:T, :] = jnp.zeros((1, E), jnp.float32)

        s_ref[...] = gat_ref[...].astype(jnp.bfloat16)

    return pl.pallas_call(
        body,
        grid=(BS // T,),
        in_specs=[
            pl.BlockSpec(memory_space=pltpu.SMEM),
            pl.BlockSpec(memory_space=_ANY),
        ],
        out_specs=pl.BlockSpec((T, E), lambda t: (t, 0)),
        out_shape=jax.ShapeDtypeStruct((BS, E), jnp.bfloat16),
        scratch_shapes=[
            pltpu.VMEM((T, E), jnp.float32),
            pltpu.SemaphoreType.DMA,
        ],
        compiler_params=pltpu.CompilerParams(
            dimension_semantics=("parallel",),
            vmem_limit_bytes=56 * 1024 * 1024,
        ),
        name="gather_next",
    )(ids_next, embed_table)


def _v_call(shifted_bf, wp_bf, BS, H, E):
    bm = min(1024, BS)
    bh = min(1024, H)
    assert BS % bm == 0 and H % bh == 0

    def body(s_ref, wp_ref, v_ref):
        v_ref[...] = jax.lax.dot_general(
            s_ref[...], wp_ref[...], (((1,), (1,)), ((), ())),
            preferred_element_type=jnp.float32,
        ).astype(jnp.bfloat16)

    return pl.pallas_call(
        body,
        grid=(BS // bm, H // bh),
        in_specs=[
            pl.BlockSpec((bm, E), lambda t, h: (t, 0)),
            pl.BlockSpec((bh, E), lambda t, h: (h, 0)),
        ],
        out_specs=pl.BlockSpec((bm, bh), lambda t, h: (t, h)),
        out_shape=jax.ShapeDtypeStruct((BS, H), jnp.bfloat16),
        compiler_params=pltpu.CompilerParams(
            dimension_semantics=("parallel", "arbitrary"),
            vmem_limit_bytes=56 * 1024 * 1024,
        ),
        name="v_proj",
    )(shifted_bf, wp_bf)


def _ttt_call(x_f32, v_bf, init_A, init_BT, B, NC, K, H, r):
    def body(x_ref, v_ref, ia_ref, ibt_ref, o_ref, xb_ref, accA, accB):
        n = pl.program_id(1)

        @pl.when(n == 0)
        def _():
            accA[...] = jnp.zeros_like(accA)
            accB[...] = jnp.zeros_like(accB)

        Zc = x_ref[...].astype(jnp.bfloat16)
        xb_ref[...] = Zc
        Vc = v_ref[...]
        iA = ia_ref[...]
        iBT = ibt_ref[...]

        A_eff = (iA - LR * accA[...]).astype(jnp.bfloat16)
        rhs2 = jnp.concatenate(
            [iBT.astype(jnp.bfloat16), accB[...].astype(jnp.bfloat16)], axis=1
        )
        p2 = jax.lax.dot_general(
            Zc, rhs2, (((1,), (0,)), ((), ())),
            preferred_element_type=jnp.float32)
        proj_in = p2[:, :r]
        mid = proj_in - LR * p2[:, r:]

        lora = jax.lax.dot_general(
            mid.astype(jnp.bfloat16), A_eff, (((1,), (1,)), ((), ())),
            preferred_element_type=jnp.float32)
        o_ref[...] = (lora * LR_SCALE).astype(jnp.bfloat16)

        dA = jax.lax.dot_general(
            Vc, proj_in.astype(jnp.bfloat16), (((0,), (0,)), ((), ())),
            preferred_element_type=jnp.float32)
        accA[...] += dA

        proj_err = jax.lax.dot_general(
            Vc, iA.astype(jnp.bfloat16), (((1,), (0,)), ((), ())),
            preferred_element_type=jnp.float32)
        dBT = jax.lax.dot_general(
            Zc, proj_err.astype(jnp.bfloat16), (((0,), (0,)), ((), ())),
            preferred_element_type=jnp.float32)
        accB[...] += dBT

    return pl.pallas_call(
        body,
        grid=(B, NC),
        in_specs=[
            pl.BlockSpec((CHUNK, K), lambda b, n: (b * NC + n, 0)),
            pl.BlockSpec((CHUNK, H), lambda b, n: (b * NC + n, 0)),
            pl.BlockSpec((H, r), lambda b, n: (0, 0)),
            pl.BlockSpec((K, r), lambda b, n: (0, 0)),
        ],
        out_specs=[
            pl.BlockSpec((CHUNK, H), lambda b, n: (b * NC + n, 0)),
            pl.BlockSpec((CHUNK, K), lambda b, n: (b * NC + n, 0)),
        ],
        out_shape=[
            jax.ShapeDtypeStruct((B * NC * CHUNK, H), jnp.bfloat16),
            jax.ShapeDtypeStruct((B * NC * CHUNK, K), jnp.bfloat16),
        ],
        scratch_shapes=[
            pltpu.VMEM((H, r), jnp.float32),
            pltpu.VMEM((K, r), jnp.float32),
        ],
        compiler_params=pltpu.CompilerParams(
            dimension_semantics=("parallel", "arbitrary"),
            vmem_limit_bytes=56 * 1024 * 1024,
        ),
        name="ttt_scan",
    )(x_f32, v_bf, init_A, init_BT)


def _base_call(x_bf, wb_bf, lora_bf, BS, K, H):
    bm = min(512, BS)
    bn = min(512, H)
    assert BS % bm == 0 and H % bn == 0

    def body(x_ref, w_ref, l_ref, o_ref):
        o_ref[...] = jax.lax.dot_general(
            x_ref[...], w_ref[...], (((1,), (1,)), ((), ())),
            preferred_element_type=jnp.float32,
        ) + l_ref[...].astype(jnp.float32)

    return pl.pallas_call(
        body,
        grid=(BS // bm, H // bn),
        in_specs=[
            pl.BlockSpec((bm, K), lambda i, j: (i, 0)),
            pl.BlockSpec((bn, K), lambda i, j: (j, 0)),
            pl.BlockSpec((bm, bn), lambda i, j: (i, j)),
        ],
        out_specs=pl.BlockSpec((bm, bn), lambda i, j: (i, j)),
        out_shape=jax.ShapeDtypeStruct((BS, H), jnp.float32),
        compiler_params=pltpu.CompilerParams(
            dimension_semantics=("parallel", "arbitrary"),
            vmem_limit_bytes=56 * 1024 * 1024,
        ),
        name="base_lora",
    )(x_bf, wb_bf, lora_bf)


def kernel(x, W_base, embed_table, W_proj, init_A, init_B, input_ids):
    B, S, K = x.shape
    H = W_base.shape[0]
    E = embed_table.shape[1]
    r = init_A.shape[1]
    BS = B * S
    NC = S // CHUNK
    assert S % CHUNK == 0

    wb_bf = W_base.astype(jnp.bfloat16)
    wp_bf = W_proj.astype(jnp.bfloat16)
    ids_next = jnp.concatenate(
        [input_ids[:, 1:], input_ids[:, :1]], axis=1
    ).reshape(BS).astype(jnp.int32)

    shifted_bf = _gather_call(ids_next, embed_table, BS, S, E)
    v_bf = _v_call(shifted_bf, wp_bf, BS, H, E)
    lora_bf, x_bf = _ttt_call(
        x.reshape(BS, K), v_bf, init_A, init_B.T, B, NC, K, H, r)
    out = _base_call(x_bf, wb_bf, lora_bf, BS, K, H)
    return out.reshape(B, S, H)


# speedup vs baseline: 1.2725x; 1.0329x over previous
import jax
import jax.numpy as jnp
from jax.experimental import pallas as pl
from jax.experimental.pallas import tpu as pltpu

CHUNK = 64
TTT_LR = 0.01
LR_SCALE = 2.0
LR = TTT_LR * LR_SCALE

_ANY = getattr(pl, "ANY", None)
if _ANY is None:
    _ANY = pltpu.ANY


def _gather_call(ids_next, embed_table, BS, S, E):
    T = min(512, S)
    assert S % T == 0 and BS % T == 0
    ratio = S // T

    def body(ids_ref, emb_ref, s_ref, gat_ref, sem):
        t = pl.program_id(0)
        base = t * T
        for i in range(T):
            pltpu.make_async_copy(
                emb_ref.at[ids_ref[base + i]], gat_ref.at[i], sem
            ).start()
        for i in range(T):
            pltpu.make_async_copy(
                emb_ref.at[ids_ref[base + i]], gat_ref.at[i], sem
            ).wait()

        @pl.when((t + 1) % ratio == 0)
        def _():
            gat_ref[T - 1:T, :] = jnp.zeros((1, E), jnp.float32)

        s_ref[...] = gat_ref[...].astype(jnp.bfloat16)

    return pl.pallas_call(
        body,
        grid=(BS // T,),
        in_specs=[
            pl.BlockSpec(memory_space=pltpu.SMEM),
            pl.BlockSpec(memory_space=_ANY),
        ],
        out_specs=pl.BlockSpec((T, E), lambda t: (t, 0)),
        out_shape=jax.ShapeDtypeStruct((BS, E), jnp.bfloat16),
        scratch_shapes=[
            pltpu.VMEM((T, E), jnp.float32),
            pltpu.SemaphoreType.DMA,
        ],
        compiler_params=pltpu.CompilerParams(
            dimension_semantics=("parallel",),
            vmem_limit_bytes=56 * 1024 * 1024,
        ),
        name="gather_next",
    )(ids_next, embed_table)


def _v_call(shifted_bf, wp_bf, W_base, BS, H, E, K):
    bm = min(512, BS)
    bh = min(1024, H)
    assert BS % bm == 0 and H % bh == 0
    tt, ht = BS // bm, H // bh
    assert H % (tt * ht) == 0
    wr = H // (tt * ht)

    def body(s_ref, wp_ref, wb_ref, v_ref, wbb_ref):
        v_ref[...] = jax.lax.dot_general(
            s_ref[...], wp_ref[...], (((1,), (1,)), ((), ())),
            preferred_element_type=jnp.float32,
        ).astype(jnp.bfloat16)
        wbb_ref[...] = wb_ref[...].astype(jnp.bfloat16)

    return pl.pallas_call(
        body,
        grid=(tt, ht),
        in_specs=[
            pl.BlockSpec((bm, E), lambda t, h: (t, 0)),
            pl.BlockSpec((bh, E), lambda t, h: (h, 0)),
            pl.BlockSpec((wr, K), lambda t, h: (t * ht + h, 0)),
        ],
        out_specs=[
            pl.BlockSpec((bm, bh), lambda t, h: (t, h)),
            pl.BlockSpec((wr, K), lambda t, h: (t * ht + h, 0)),
        ],
        out_shape=[
            jax.ShapeDtypeStruct((BS, H), jnp.bfloat16),
            jax.ShapeDtypeStruct((H, K), jnp.bfloat16),
        ],
        compiler_params=pltpu.CompilerParams(
            dimension_semantics=("parallel", "arbitrary"),
            vmem_limit_bytes=56 * 1024 * 1024,
        ),
        name="v_proj",
    )(shifted_bf, wp_bf, W_base)


def _ttt_call(x_f32, v_bf, init_A, init_BT, B, NC, K, H, r):
    def body(x_ref, v_ref, ia_ref, ibt_ref, o_ref, xb_ref, accA, accB):
        n = pl.program_id(1)

        @pl.when(n == 0)
        def _():
            accA[...] = jnp.zeros_like(accA)
            accB[...] = jnp.zeros_like(accB)

        Zc = x_ref[...].astype(jnp.bfloat16)
        xb_ref[...] = Zc
        Vc = v_ref[...]
        iA = ia_ref[...]
        iBT = ibt_ref[...]

        A_eff = (iA - LR * accA[...]).astype(jnp.bfloat16)
        rhs2 = jnp.concatenate(
            [iBT.astype(jnp.bfloat16), accB[...].astype(jnp.bfloat16)], axis=1
        )
        p2 = jax.lax.dot_general(
            Zc, rhs2, (((1,), (0,)), ((), ())),
            preferred_element_type=jnp.float32)
        proj_in = p2[:, :r]
        mid = proj_in - LR * p2[:, r:]

        lora = jax.lax.dot_general(
            mid.astype(jnp.bfloat16), A_eff, (((1,), (1,)), ((), ())),
            preferred_element_type=jnp.float32)
        o_ref[...] = (lora * LR_SCALE).astype(jnp.bfloat16)

        dA = jax.lax.dot_general(
            Vc, proj_in.astype(jnp.bfloat16), (((0,), (0,)), ((), ())),
            preferred_element_type=jnp.float32)
        accA[...] += dA

        proj_err = jax.lax.dot_general(
            Vc, iA.astype(jnp.bfloat16), (((1,), (0,)), ((), ())),
            preferred_element_type=jnp.float32)
        dBT = jax.lax.dot_general(
            Zc, proj_err.astype(jnp.bfloat16), (((0,), (0,)), ((), ())),
            preferred_element_type=jnp.float32)
        accB[...] += dBT

    return pl.pallas_call(
        body,
        grid=(B, NC),
        in_specs=[
            pl.BlockSpec((CHUNK, K), lambda b, n: (b * NC + n, 0)),
            pl.BlockSpec((CHUNK, H), lambda b, n: (b * NC + n, 0)),
            pl.BlockSpec((H, r), lambda b, n: (0, 0)),
            pl.BlockSpec((K, r), lambda b, n: (0, 0)),
        ],
        out_specs=[
            pl.BlockSpec((CHUNK, H), lambda b, n: (b * NC + n, 0)),
            pl.BlockSpec((CHUNK, K), lambda b, n: (b * NC + n, 0)),
        ],
        out_shape=[
            jax.ShapeDtypeStruct((B * NC * CHUNK, H), jnp.bfloat16),
            jax.ShapeDtypeStruct((B * NC * CHUNK, K), jnp.bfloat16),
        ],
        scratch_shapes=[
            pltpu.VMEM((H, r), jnp.float32),
            pltpu.VMEM((K, r), jnp.float32),
        ],
        compiler_params=pltpu.CompilerParams(
            dimension_semantics=("parallel", "arbitrary"),
            vmem_limit_bytes=56 * 1024 * 1024,
        ),
        name="ttt_scan",
    )(x_f32, v_bf, init_A, init_BT)


def _base_call(x_bf, wb_bf, lora_bf, BS, K, H):
    bm = min(512, BS)
    bn = min(512, H)
    assert BS % bm == 0 and H % bn == 0

    def body(x_ref, w_ref, l_ref, o_ref):
        o_ref[...] = jax.lax.dot_general(
            x_ref[...], w_ref[...], (((1,), (1,)), ((), ())),
            preferred_element_type=jnp.float32,
        ) + l_ref[...].astype(jnp.float32)

    return pl.pallas_call(
        body,
        grid=(BS // bm, H // bn),
        in_specs=[
            pl.BlockSpec((bm, K), lambda i, j: (i, 0)),
            pl.BlockSpec((bn, K), lambda i, j: (j, 0)),
            pl.BlockSpec((bm, bn), lambda i, j: (i, j)),
        ],
        out_specs=pl.BlockSpec((bm, bn), lambda i, j: (i, j)),
        out_shape=jax.ShapeDtypeStruct((BS, H), jnp.float32),
        compiler_params=pltpu.CompilerParams(
            dimension_semantics=("parallel", "arbitrary"),
            vmem_limit_bytes=56 * 1024 * 1024,
        ),
        name="base_lora",
    )(x_bf, wb_bf, lora_bf)


def kernel(x, W_base, embed_table, W_proj, init_A, init_B, input_ids):
    B, S, K = x.shape
    H = W_base.shape[0]
    E = embed_table.shape[1]
    r = init_A.shape[1]
    BS = B * S
    NC = S // CHUNK
    assert S % CHUNK == 0

    wp_bf = W_proj.astype(jnp.bfloat16)
    ids_next = jnp.concatenate(
        [input_ids[:, 1:], input_ids[:, :1]], axis=1
    ).reshape(BS).astype(jnp.int32)

    shifted_bf = _gather_call(ids_next, embed_table, BS, S, E)
    v_bf, wb_bf = _v_call(shifted_bf, wp_bf, W_base, BS, H, E, K)
    lora_bf, x_bf = _ttt_call(
        x.reshape(BS, K), v_bf, init_A, init_B.T, B, NC, K, H, r)
    out = _base_call(x_bf, wb_bf, lora_bf, BS, K, H)
    return out.reshape(B, S, H)
